# baseline (speedup 1.0000x reference)
import dataclasses
import functools

import jax
import jax.numpy as jnp
from jax import lax
from jax.experimental import pallas as pl
from jax.experimental.pallas import tpu as pltpu
from jax.experimental.pallas import tpu_sc as plsc

B = 4096
L = 200
D = 128
NC = 2
NS = 16
NW = NC * NS
BL = B // NW
S0, S1 = 104, 96
NCHUNK = D // 16
DP = D // 2


def _take(vec, idx):
    dnums = lax.GatherDimensionNumbers(
        offset_dims=(), collapsed_slice_dims=(0,), start_index_map=(0,))
    return lax.gather(vec, idx[:, None], dnums, slice_sizes=(1,),
                      mode=lax.GatherScatterMode.PROMISE_IN_BOUNDS)


def _lane_sum_bcast(acc):
    iota = lax.iota(jnp.int32, 16)
    for sh in (1, 2, 4, 8):
        acc = acc + _take(acc, lax.bitwise_xor(iota, sh))
    return acc


def _mf_sc(user, item_flat, user_table, item_table, user_bias, item_bias):
    mesh = plsc.VectorSubcoreMesh(core_axis_name="c", subcore_axis_name="s")
    cp = pltpu.CompilerParams(use_tc_tiling_on_sc=False)
    if "needs_layout_passes" in pltpu.CompilerParams.__dataclass_fields__:
        cp = dataclasses.replace(cp, needs_layout_passes=False)

    @functools.partial(
        pl.kernel,
        out_type=jax.ShapeDtypeStruct((B, L), jnp.float32),
        mesh=mesh,
        compiler_params=cp,
        scratch_types=[
            pltpu.VMEM((BL,), jnp.int32),
            pltpu.VMEM((BL * L,), jnp.int32),
            pltpu.VMEM((BL, D), jnp.float32),
            pltpu.VMEM((BL,), jnp.float32),
            pltpu.VMEM((L, DP), jnp.uint32),
            pltpu.VMEM((L, DP), jnp.uint32),
            pltpu.VMEM((L,), jnp.float32),
            pltpu.VMEM((L,), jnp.float32),
            pltpu.VMEM((BL, L), jnp.float32),
            pltpu.SemaphoreType.DMA,
            pltpu.SemaphoreType.DMA,
            pltpu.SemaphoreType.DMA,
        ],
    )
    def k(user_hbm, item_hbm, utab_hbm, itab_hbm, ubias_hbm, ibias_hbm,
          out_hbm, uidx_v, idx_v, urows_v, ub_v, rows_a, rows_b, ib_a, ib_b,
          out_v, sem_a, sem_b, sem0):
        wid = lax.axis_index("s") * NC + lax.axis_index("c")
        base = wid * BL

        pltpu.sync_copy(user_hbm.at[pl.ds(base, BL)], uidx_v)
        pltpu.sync_copy(item_hbm.at[pl.ds(base * L, BL * L)], idx_v)
        pltpu.async_copy(utab_hbm.at[uidx_v], urows_v, sem0).wait()
        pltpu.async_copy(ubias_hbm.at[uidx_v], ub_v, sem0).wait()

        def idx_views(b):
            o = pl.multiple_of(b * L, 8)
            return (idx_v.at[pl.ds(o, S0)],
                    idx_v.at[pl.ds(pl.multiple_of(b * L + S0, 8), S1)])

        def fire(b, rows, ib, sem):
            i1, i2 = idx_views(b)
            pltpu.async_copy(itab_hbm.at[i1], rows.at[pl.ds(0, S0)], sem)
            pltpu.async_copy(itab_hbm.at[i2], rows.at[pl.ds(S0, S1)], sem)
            pltpu.async_copy(ibias_hbm.at[i1], ib.at[pl.ds(0, S0)], sem)
            pltpu.async_copy(ibias_hbm.at[i2], ib.at[pl.ds(S0, S1)], sem)

        def drain(b, rows, ib, sem):
            i1, i2 = idx_views(b)
            pltpu.make_async_copy(itab_hbm.at[i1], rows.at[pl.ds(0, S0)],
                                  sem).wait()
            pltpu.make_async_copy(itab_hbm.at[i2], rows.at[pl.ds(S0, S1)],
                                  sem).wait()
            pltpu.make_async_copy(ibias_hbm.at[i1], ib.at[pl.ds(0, S0)],
                                  sem).wait()
            pltpu.make_async_copy(ibias_hbm.at[i2], ib.at[pl.ds(S0, S1)],
                                  sem).wait()

        lane_iota = lax.iota(jnp.int32, 16)

        def compute(b, rows, ib):
            u = [urows_v[b, pl.ds(16 * c, 16)] for c in range(NCHUNK)]
            ubf = [plsc.pack(u[c], u[c + 4],
                             format=plsc.PackFormat.INTERLEAVED)
                   for c in range(DP // 16)]
            ub_chunk = ub_v[pl.ds((b // 16) * 16, 16)]
            ubs = _take(ub_chunk, jnp.full((16,), lax.rem(b, 16), jnp.int32))

            @pl.loop(0, 13)
            def _(g):
                off = jnp.minimum(16 * g, L - 16)
                out16 = jnp.zeros((16,), jnp.float32)
                for j in range(16):
                    row = off + j
                    bf = [plsc.bitcast(rows[row, pl.ds(16 * c, 16)],
                                       jnp.bfloat16)
                          for c in range(DP // 16)]
                    acc_bf = bf[0] * ubf[0] + bf[1] * ubf[1]
                    acc_bf = acc_bf + bf[2] * ubf[2]
                    acc_bf = acc_bf + bf[3] * ubf[3]
                    ev, od = plsc.unpack(
                        acc_bf, format=plsc.PackFormat.INTERLEAVED)
                    acc = ev + od
                    tot = _take(jnp.cumsum(acc), jnp.full((16,), 15,
                                                          jnp.int32))
                    out16 = jnp.where(lane_iota == j, tot, out16)
                out16 = out16 + ib[pl.ds(off, 16)] + ubs
                out_v[b, pl.ds(off, 16)] = out16

        fire(0, rows_a, ib_a, sem_a)

        @pl.loop(0, BL, step=2)
        def _(b):
            fire(b + 1, rows_b, ib_b, sem_b)
            drain(b, rows_a, ib_a, sem_a)
            compute(b, rows_a, ib_a)

            @pl.when(b + 2 < BL)
            def _():
                fire(b + 2, rows_a, ib_a, sem_a)

            drain(b + 1, rows_b, ib_b, sem_b)
            compute(b + 1, rows_b, ib_b)

        pltpu.sync_copy(out_v, out_hbm.at[pl.ds(base, BL)])

    return k(user, item_flat, user_table, item_table, user_bias, item_bias)


V = 100000
RPW = V // NW
CH = 125
NCHK = RPW // CH


def _pack_sc(item_table):
    mesh = plsc.VectorSubcoreMesh(core_axis_name="c", subcore_axis_name="s")
    cp = pltpu.CompilerParams(use_tc_tiling_on_sc=False)
    if "needs_layout_passes" in pltpu.CompilerParams.__dataclass_fields__:
        cp = dataclasses.replace(cp, needs_layout_passes=False)

    @functools.partial(
        pl.kernel,
        out_type=jax.ShapeDtypeStruct((V, DP), jnp.uint32),
        mesh=mesh,
        compiler_params=cp,
        scratch_types=[
            pltpu.VMEM((CH, D), jnp.float32),
            pltpu.VMEM((CH, D), jnp.float32),
            pltpu.VMEM((CH, DP), jnp.uint32),
            pltpu.VMEM((CH, DP), jnp.uint32),
            pltpu.SemaphoreType.DMA,
            pltpu.SemaphoreType.DMA,
            pltpu.SemaphoreType.DMA,
            pltpu.SemaphoreType.DMA,
        ],
    )
    def k(tab_hbm, out_hbm, in_a, in_b, out_a, out_b, isem_a, isem_b,
          osem_a, osem_b):
        wid = lax.axis_index("s") * NC + lax.axis_index("c")
        rbase = wid * RPW

        def fire_in(c, buf, sem):
            pltpu.async_copy(tab_hbm.at[pl.ds(rbase + c * CH, CH)], buf, sem)

        def drain_in(c, buf, sem):
            pltpu.make_async_copy(tab_hbm.at[pl.ds(rbase + c * CH, CH)],
                                  buf, sem).wait()

        def fire_out(c, buf, sem):
            pltpu.async_copy(buf, out_hbm.at[pl.ds(rbase + c * CH, CH)], sem)

        def drain_out(c, buf, sem):
            pltpu.make_async_copy(buf, out_hbm.at[pl.ds(rbase + c * CH, CH)],
                                  sem).wait()

        half = jnp.uint32(0x8000)
        himask = jnp.uint32(0xFFFF0000)

        def compute(inb, outb):
            @pl.loop(0, CH, step=5)
            def _(r0):
                for dr in range(5):
                    r = r0 + dr
                    for g in range(DP // 16):
                        lo = plsc.bitcast(inb[r, pl.ds(16 * g, 16)],
                                          jnp.uint32)
                        hi = plsc.bitcast(inb[r, pl.ds(DP + 16 * g, 16)],
                                          jnp.uint32)
                        outb[r, pl.ds(16 * g, 16)] = (
                            ((lo + half) >> 16) | ((hi + half) & himask))

        fire_in(0, in_a, isem_a)

        @pl.loop(0, NCHK - 1, step=2)
        def _(c):
            fire_in(c + 1, in_b, isem_b)
            drain_in(c, in_a, isem_a)

            @pl.when(c > 0)
            def _():
                drain_out(c - 2, out_a, osem_a)

            compute(in_a, out_a)
            fire_out(c, out_a, osem_a)

            @pl.when(c + 2 < NCHK)
            def _():
                fire_in(c + 2, in_a, isem_a)

            drain_in(c + 1, in_b, isem_b)

            @pl.when(c > 0)
            def _():
                drain_out(c - 1, out_b, osem_b)

            compute(in_b, out_b)
            fire_out(c + 1, out_b, osem_b)

        drain_in(NCHK - 1, in_a, isem_a)
        drain_out(NCHK - 3, out_a, osem_a)
        compute(in_a, out_a)
        fire_out(NCHK - 1, out_a, osem_a)
        drain_out(NCHK - 2, out_b, osem_b)
        drain_out(NCHK - 1, out_a, osem_a)

    return k(item_table)


def _pack_tc(item_table):
    blk = 2000

    def body(x_ref, o_ref):
        xi = lax.bitcast_convert_type(x_ref[...], jnp.uint32)
        half = jnp.uint32(0x8000)
        lo = (xi[:, :DP] + half) >> 16
        hi = (xi[:, DP:] + half) & jnp.uint32(0xFFFF0000)
        o_ref[...] = lo | hi

    return pl.pallas_call(
        body,
        out_shape=jax.ShapeDtypeStruct((V, DP), jnp.uint32),
        grid=(V // blk,),
        in_specs=[pl.BlockSpec((blk, D), lambda i: (i, 0))],
        out_specs=pl.BlockSpec((blk, DP), lambda i: (i, 0)),
    )(item_table)


def kernel(user, item, user_table, item_table, user_bias, item_bias):
    item_flat = item.reshape(B * L).astype(jnp.int32)
    user = user.astype(jnp.int32)
    itab_pk = _pack_tc(item_table)
    return _mf_sc(user, item_flat, user_table, itab_pk, user_bias,
                  item_bias)


# speedup vs baseline: 1.0315x; 1.0315x over previous
import dataclasses
import functools

import jax
import jax.numpy as jnp
from jax import lax
from jax.experimental import pallas as pl
from jax.experimental.pallas import tpu as pltpu
from jax.experimental.pallas import tpu_sc as plsc

B = 4096
L = 200
D = 128
NC = 2
NS = 16
NW = NC * NS
BL = B // NW
S0, S1 = 104, 96
NCHUNK = D // 16
DP = D // 2


def _take(vec, idx):
    dnums = lax.GatherDimensionNumbers(
        offset_dims=(), collapsed_slice_dims=(0,), start_index_map=(0,))
    return lax.gather(vec, idx[:, None], dnums, slice_sizes=(1,),
                      mode=lax.GatherScatterMode.PROMISE_IN_BOUNDS)


def _lane_sum_bcast(acc):
    iota = lax.iota(jnp.int32, 16)
    for sh in (1, 2, 4, 8):
        acc = acc + _take(acc, lax.bitwise_xor(iota, sh))
    return acc


def _mf_sc(user, item_flat, user_table, item_table, user_bias, item_bias):
    mesh = plsc.VectorSubcoreMesh(core_axis_name="c", subcore_axis_name="s")
    cp = pltpu.CompilerParams(use_tc_tiling_on_sc=False)
    if "needs_layout_passes" in pltpu.CompilerParams.__dataclass_fields__:
        cp = dataclasses.replace(cp, needs_layout_passes=False)

    @functools.partial(
        pl.kernel,
        out_type=jax.ShapeDtypeStruct((B, L), jnp.float32),
        mesh=mesh,
        compiler_params=cp,
        scratch_types=[
            pltpu.VMEM((BL,), jnp.int32),
            pltpu.VMEM((BL * L,), jnp.int32),
            pltpu.VMEM((BL, D), jnp.float32),
            pltpu.VMEM((BL,), jnp.float32),
            pltpu.VMEM((L, DP), jnp.uint32),
            pltpu.VMEM((L, DP), jnp.uint32),
            pltpu.VMEM((L,), jnp.float32),
            pltpu.VMEM((L,), jnp.float32),
            pltpu.VMEM((BL, L), jnp.float32),
            pltpu.SemaphoreType.DMA,
            pltpu.SemaphoreType.DMA,
            pltpu.SemaphoreType.DMA,
        ],
    )
    def k(user_hbm, item_hbm, utab_hbm, itab_hbm, ubias_hbm, ibias_hbm,
          out_hbm, uidx_v, idx_v, urows_v, ub_v, rows_a, rows_b, ib_a, ib_b,
          out_v, sem_a, sem_b, sem0):
        wid = lax.axis_index("s") * NC + lax.axis_index("c")
        base = wid * BL

        pltpu.sync_copy(user_hbm.at[pl.ds(base, BL)], uidx_v)
        pltpu.sync_copy(item_hbm.at[pl.ds(base * L, BL * L)], idx_v)
        pltpu.async_copy(utab_hbm.at[uidx_v], urows_v, sem0).wait()
        pltpu.async_copy(ubias_hbm.at[uidx_v], ub_v, sem0).wait()

        def idx_views(b):
            o = pl.multiple_of(b * L, 8)
            return (idx_v.at[pl.ds(o, S0)],
                    idx_v.at[pl.ds(pl.multiple_of(b * L + S0, 8), S1)])

        def fire(b, rows, ib, sem):
            i1, i2 = idx_views(b)
            pltpu.async_copy(itab_hbm.at[i1], rows.at[pl.ds(0, S0)], sem)
            pltpu.async_copy(itab_hbm.at[i2], rows.at[pl.ds(S0, S1)], sem)
            pltpu.async_copy(ibias_hbm.at[i1], ib.at[pl.ds(0, S0)], sem)
            pltpu.async_copy(ibias_hbm.at[i2], ib.at[pl.ds(S0, S1)], sem)

        def drain(b, rows, ib, sem):
            i1, i2 = idx_views(b)
            pltpu.make_async_copy(itab_hbm.at[i1], rows.at[pl.ds(0, S0)],
                                  sem).wait()
            pltpu.make_async_copy(itab_hbm.at[i2], rows.at[pl.ds(S0, S1)],
                                  sem).wait()
            pltpu.make_async_copy(ibias_hbm.at[i1], ib.at[pl.ds(0, S0)],
                                  sem).wait()
            pltpu.make_async_copy(ibias_hbm.at[i2], ib.at[pl.ds(S0, S1)],
                                  sem).wait()

        lane_iota = lax.iota(jnp.int32, 16)

        def compute(b, rows, ib):
            u = [urows_v[b, pl.ds(16 * c, 16)] for c in range(NCHUNK)]
            ubf = [plsc.pack(u[c], u[c + 4],
                             format=plsc.PackFormat.INTERLEAVED)
                   for c in range(DP // 16)]
            ub_chunk = ub_v[pl.ds((b // 16) * 16, 16)]
            ubs = _take(ub_chunk, jnp.full((16,), lax.rem(b, 16), jnp.int32))

            @pl.loop(0, 13)
            def _(g):
                off = jnp.minimum(16 * g, L - 16)
                out16 = jnp.zeros((16,), jnp.float32)
                for j in range(16):
                    row = off + j
                    bf = [plsc.bitcast(rows[row, pl.ds(16 * c, 16)],
                                       jnp.bfloat16)
                          for c in range(DP // 16)]
                    acc_bf = bf[0] * ubf[0] + bf[1] * ubf[1]
                    acc_bf = acc_bf + bf[2] * ubf[2]
                    acc_bf = acc_bf + bf[3] * ubf[3]
                    ev, od = plsc.unpack(
                        acc_bf, format=plsc.PackFormat.INTERLEAVED)
                    acc = ev + od
                    tot = _take(jnp.cumsum(acc), jnp.full((16,), 15,
                                                          jnp.int32))
                    out16 = jnp.where(lane_iota == j, tot, out16)
                out16 = out16 + ib[pl.ds(off, 16)] + ubs
                out_v[b, pl.ds(off, 16)] = out16

        fire(0, rows_a, ib_a, sem_a)

        @pl.loop(0, BL, step=2)
        def _(b):
            fire(b + 1, rows_b, ib_b, sem_b)
            drain(b, rows_a, ib_a, sem_a)
            compute(b, rows_a, ib_a)

            @pl.when(b + 2 < BL)
            def _():
                fire(b + 2, rows_a, ib_a, sem_a)

            drain(b + 1, rows_b, ib_b, sem_b)
            compute(b + 1, rows_b, ib_b)

        pltpu.sync_copy(out_v, out_hbm.at[pl.ds(base, BL)])

    return k(user, item_flat, user_table, item_table, user_bias, item_bias)


V = 100000
RPW = V // NW
CH = 125
NCHK = RPW // CH


def _pack_sc(item_table):
    mesh = plsc.VectorSubcoreMesh(core_axis_name="c", subcore_axis_name="s")
    cp = pltpu.CompilerParams(use_tc_tiling_on_sc=False)
    if "needs_layout_passes" in pltpu.CompilerParams.__dataclass_fields__:
        cp = dataclasses.replace(cp, needs_layout_passes=False)

    @functools.partial(
        pl.kernel,
        out_type=jax.ShapeDtypeStruct((V, DP), jnp.uint32),
        mesh=mesh,
        compiler_params=cp,
        scratch_types=[
            pltpu.VMEM((CH, D), jnp.float32),
            pltpu.VMEM((CH, D), jnp.float32),
            pltpu.VMEM((CH, DP), jnp.uint32),
            pltpu.VMEM((CH, DP), jnp.uint32),
            pltpu.SemaphoreType.DMA,
            pltpu.SemaphoreType.DMA,
            pltpu.SemaphoreType.DMA,
            pltpu.SemaphoreType.DMA,
        ],
    )
    def k(tab_hbm, out_hbm, in_a, in_b, out_a, out_b, isem_a, isem_b,
          osem_a, osem_b):
        wid = lax.axis_index("s") * NC + lax.axis_index("c")
        rbase = wid * RPW

        def fire_in(c, buf, sem):
            pltpu.async_copy(tab_hbm.at[pl.ds(rbase + c * CH, CH)], buf, sem)

        def drain_in(c, buf, sem):
            pltpu.make_async_copy(tab_hbm.at[pl.ds(rbase + c * CH, CH)],
                                  buf, sem).wait()

        def fire_out(c, buf, sem):
            pltpu.async_copy(buf, out_hbm.at[pl.ds(rbase + c * CH, CH)], sem)

        def drain_out(c, buf, sem):
            pltpu.make_async_copy(buf, out_hbm.at[pl.ds(rbase + c * CH, CH)],
                                  sem).wait()

        half = jnp.uint32(0x8000)
        himask = jnp.uint32(0xFFFF0000)

        def compute(inb, outb):
            @pl.loop(0, CH, step=5)
            def _(r0):
                for dr in range(5):
                    r = r0 + dr
                    for g in range(DP // 16):
                        lo = plsc.bitcast(inb[r, pl.ds(16 * g, 16)],
                                          jnp.uint32)
                        hi = plsc.bitcast(inb[r, pl.ds(DP + 16 * g, 16)],
                                          jnp.uint32)
                        outb[r, pl.ds(16 * g, 16)] = (
                            ((lo + half) >> 16) | ((hi + half) & himask))

        fire_in(0, in_a, isem_a)

        @pl.loop(0, NCHK - 1, step=2)
        def _(c):
            fire_in(c + 1, in_b, isem_b)
            drain_in(c, in_a, isem_a)

            @pl.when(c > 0)
            def _():
                drain_out(c - 2, out_a, osem_a)

            compute(in_a, out_a)
            fire_out(c, out_a, osem_a)

            @pl.when(c + 2 < NCHK)
            def _():
                fire_in(c + 2, in_a, isem_a)

            drain_in(c + 1, in_b, isem_b)

            @pl.when(c > 0)
            def _():
                drain_out(c - 1, out_b, osem_b)

            compute(in_b, out_b)
            fire_out(c + 1, out_b, osem_b)

        drain_in(NCHK - 1, in_a, isem_a)
        drain_out(NCHK - 3, out_a, osem_a)
        compute(in_a, out_a)
        fire_out(NCHK - 1, out_a, osem_a)
        drain_out(NCHK - 2, out_b, osem_b)
        drain_out(NCHK - 1, out_a, osem_a)

    return k(item_table)


def _pack_tc(item_table):
    blk = 1000

    def body(x1_ref, x2_ref, o_ref):
        half = jnp.uint32(0x8000)
        himask = jnp.uint32(0xFFFF0000)
        x1 = lax.bitcast_convert_type(x1_ref[...], jnp.uint32)
        x2 = lax.bitcast_convert_type(x2_ref[...], jnp.uint32)
        o_ref[:, :DP] = ((x1[:, :DP] + half) >> 16) | ((x1[:, DP:] + half)
                                                       & himask)
        o_ref[:, DP:] = ((x2[:, :DP] + half) >> 16) | ((x2[:, DP:] + half)
                                                       & himask)

    return pl.pallas_call(
        body,
        out_shape=jax.ShapeDtypeStruct((V // 2, D), jnp.uint32),
        grid=(V // 2 // blk,),
        in_specs=[pl.BlockSpec((blk, D), lambda i: (i, 0)),
                  pl.BlockSpec((blk, D), lambda i: (i + V // 2 // blk, 0))],
        out_specs=pl.BlockSpec((blk, D), lambda i: (i, 0)),
    )(item_table, item_table)


def kernel(user, item, user_table, item_table, user_bias, item_bias):
    user = user.astype(jnp.int32)
    item = item.astype(jnp.int32)
    item_flat = jnp.where(item < V // 2, item * 2,
                          item * 2 - (V - 1)).reshape(B * L)
    ibias_r = item_bias.reshape(2, V // 2).transpose(1, 0).reshape(V)
    itab_pk = _pack_tc(item_table).reshape(V, DP)
    return _mf_sc(user, item_flat, user_table, itab_pk, user_bias,
                  ibias_r)


# speedup vs baseline: 1.0902x; 1.0569x over previous
import dataclasses
import functools

import jax
import jax.numpy as jnp
from jax import lax
from jax.experimental import pallas as pl
from jax.experimental.pallas import tpu as pltpu
from jax.experimental.pallas import tpu_sc as plsc

B = 4096
L = 200
D = 128
NC = 2
NS = 16
NW = NC * NS
BL = B // NW
S0, S1 = 104, 96
NCHUNK = D // 16
DP = D // 2


def _take(vec, idx):
    dnums = lax.GatherDimensionNumbers(
        offset_dims=(), collapsed_slice_dims=(0,), start_index_map=(0,))
    return lax.gather(vec, idx[:, None], dnums, slice_sizes=(1,),
                      mode=lax.GatherScatterMode.PROMISE_IN_BOUNDS)


def _lane_sum_bcast(acc):
    iota = lax.iota(jnp.int32, 16)
    for sh in (1, 2, 4, 8):
        acc = acc + _take(acc, lax.bitwise_xor(iota, sh))
    return acc


def _mf_sc(user, item_flat, user_table, item_table, user_bias, item_bias):
    mesh = plsc.VectorSubcoreMesh(core_axis_name="c", subcore_axis_name="s")
    cp = pltpu.CompilerParams(use_tc_tiling_on_sc=False)
    if "needs_layout_passes" in pltpu.CompilerParams.__dataclass_fields__:
        cp = dataclasses.replace(cp, needs_layout_passes=False)

    @functools.partial(
        pl.kernel,
        out_type=jax.ShapeDtypeStruct((B, L), jnp.float32),
        mesh=mesh,
        compiler_params=cp,
        scratch_types=[
            pltpu.VMEM((BL,), jnp.int32),
            pltpu.VMEM((BL * L,), jnp.int32),
            pltpu.VMEM((BL, D), jnp.float32),
            pltpu.VMEM((BL,), jnp.float32),
            pltpu.VMEM((L, DP), jnp.uint32),
            pltpu.VMEM((L, DP), jnp.uint32),
            pltpu.VMEM((L,), jnp.float32),
            pltpu.VMEM((L,), jnp.float32),
            pltpu.VMEM((BL, L), jnp.float32),
            pltpu.SemaphoreType.DMA,
            pltpu.SemaphoreType.DMA,
            pltpu.SemaphoreType.DMA,
        ],
    )
    def k(user_hbm, item_hbm, utab_hbm, itab_hbm, ubias_hbm, ibias_hbm,
          out_hbm, uidx_v, idx_v, urows_v, ub_v, rows_a, rows_b, ib_a, ib_b,
          out_v, sem_a, sem_b, sem0):
        wid = lax.axis_index("s") * NC + lax.axis_index("c")
        base = wid * BL

        pltpu.sync_copy(user_hbm.at[pl.ds(base, BL)], uidx_v)
        pltpu.sync_copy(item_hbm.at[pl.ds(base * L, BL * L)], idx_v)
        pltpu.async_copy(utab_hbm.at[uidx_v], urows_v, sem0).wait()
        pltpu.async_copy(ubias_hbm.at[uidx_v], ub_v, sem0).wait()

        def idx_views(b):
            o = pl.multiple_of(b * L, 8)
            return (idx_v.at[pl.ds(o, S0)],
                    idx_v.at[pl.ds(pl.multiple_of(b * L + S0, 8), S1)])

        def fire(b, rows, ib, sem):
            i1, i2 = idx_views(b)
            pltpu.async_copy(itab_hbm.at[i1], rows.at[pl.ds(0, S0)], sem)
            pltpu.async_copy(itab_hbm.at[i2], rows.at[pl.ds(S0, S1)], sem)
            pltpu.async_copy(ibias_hbm.at[i1], ib.at[pl.ds(0, S0)], sem)
            pltpu.async_copy(ibias_hbm.at[i2], ib.at[pl.ds(S0, S1)], sem)

        def drain(b, rows, ib, sem):
            i1, i2 = idx_views(b)
            pltpu.make_async_copy(itab_hbm.at[i1], rows.at[pl.ds(0, S0)],
                                  sem).wait()
            pltpu.make_async_copy(itab_hbm.at[i2], rows.at[pl.ds(S0, S1)],
                                  sem).wait()
            pltpu.make_async_copy(ibias_hbm.at[i1], ib.at[pl.ds(0, S0)],
                                  sem).wait()
            pltpu.make_async_copy(ibias_hbm.at[i2], ib.at[pl.ds(S0, S1)],
                                  sem).wait()

        lane_iota = lax.iota(jnp.int32, 16)

        def compute(b, rows, ib):
            u = [urows_v[b, pl.ds(16 * c, 16)] for c in range(NCHUNK)]
            ubf = [plsc.pack(u[c], u[c + 4],
                             format=plsc.PackFormat.INTERLEAVED)
                   for c in range(DP // 16)]
            ub_chunk = ub_v[pl.ds((b // 16) * 16, 16)]
            ubs = _take(ub_chunk, jnp.full((16,), lax.rem(b, 16), jnp.int32))

            @pl.loop(0, 13)
            def _(g):
                off = jnp.minimum(16 * g, L - 16)
                out16 = jnp.zeros((16,), jnp.float32)
                for j in range(16):
                    row = off + j
                    bf = [plsc.bitcast(rows[row, pl.ds(16 * c, 16)],
                                       jnp.bfloat16)
                          for c in range(DP // 16)]
                    acc_bf = bf[0] * ubf[0] + bf[1] * ubf[1]
                    acc_bf = acc_bf + bf[2] * ubf[2]
                    acc_bf = acc_bf + bf[3] * ubf[3]
                    ev, od = plsc.unpack(
                        acc_bf, format=plsc.PackFormat.INTERLEAVED)
                    acc = ev + od
                    tot = _take(jnp.cumsum(acc), jnp.full((16,), 15,
                                                          jnp.int32))
                    out16 = jnp.where(lane_iota == j, tot, out16)
                out16 = out16 + ib[pl.ds(off, 16)] + ubs
                out_v[b, pl.ds(off, 16)] = out16

        fire(0, rows_a, ib_a, sem_a)

        @pl.loop(0, BL, step=2)
        def _(b):
            fire(b + 1, rows_b, ib_b, sem_b)
            drain(b, rows_a, ib_a, sem_a)
            compute(b, rows_a, ib_a)

            @pl.when(b + 2 < BL)
            def _():
                fire(b + 2, rows_a, ib_a, sem_a)

            drain(b + 1, rows_b, ib_b, sem_b)
            compute(b + 1, rows_b, ib_b)

        pltpu.sync_copy(out_v, out_hbm.at[pl.ds(base, BL)])

    return k(user, item_flat, user_table, item_table, user_bias, item_bias)


V = 100000
RPW = V // NW
CH = 125
NCHK = RPW // CH


def _pack_sc(item_table):
    mesh = plsc.VectorSubcoreMesh(core_axis_name="c", subcore_axis_name="s")
    cp = pltpu.CompilerParams(use_tc_tiling_on_sc=False)
    if "needs_layout_passes" in pltpu.CompilerParams.__dataclass_fields__:
        cp = dataclasses.replace(cp, needs_layout_passes=False)

    @functools.partial(
        pl.kernel,
        out_type=jax.ShapeDtypeStruct((V, DP), jnp.uint32),
        mesh=mesh,
        compiler_params=cp,
        scratch_types=[
            pltpu.VMEM((CH, D), jnp.float32),
            pltpu.VMEM((CH, D), jnp.float32),
            pltpu.VMEM((CH, DP), jnp.uint32),
            pltpu.VMEM((CH, DP), jnp.uint32),
            pltpu.SemaphoreType.DMA,
            pltpu.SemaphoreType.DMA,
            pltpu.SemaphoreType.DMA,
            pltpu.SemaphoreType.DMA,
        ],
    )
    def k(tab_hbm, out_hbm, in_a, in_b, out_a, out_b, isem_a, isem_b,
          osem_a, osem_b):
        wid = lax.axis_index("s") * NC + lax.axis_index("c")
        rbase = wid * RPW

        H1 = 64

        def fire_in(c, buf, sem):
            r0 = rbase + c * CH
            pltpu.async_copy(tab_hbm.at[pl.ds(r0, H1)],
                             buf.at[pl.ds(0, H1)], sem)
            pltpu.async_copy(tab_hbm.at[pl.ds(r0 + H1, CH - H1)],
                             buf.at[pl.ds(H1, CH - H1)], sem)

        def drain_in(c, buf, sem):
            r0 = rbase + c * CH
            pltpu.make_async_copy(tab_hbm.at[pl.ds(r0, H1)],
                                  buf.at[pl.ds(0, H1)], sem).wait()
            pltpu.make_async_copy(tab_hbm.at[pl.ds(r0 + H1, CH - H1)],
                                  buf.at[pl.ds(H1, CH - H1)], sem).wait()

        def fire_out(c, buf, sem):
            r0 = rbase + c * CH
            pltpu.async_copy(buf.at[pl.ds(0, H1)],
                             out_hbm.at[pl.ds(r0, H1)], sem)
            pltpu.async_copy(buf.at[pl.ds(H1, CH - H1)],
                             out_hbm.at[pl.ds(r0 + H1, CH - H1)], sem)

        def drain_out(c, buf, sem):
            r0 = rbase + c * CH
            pltpu.make_async_copy(buf.at[pl.ds(0, H1)],
                                  out_hbm.at[pl.ds(r0, H1)], sem).wait()
            pltpu.make_async_copy(buf.at[pl.ds(H1, CH - H1)],
                                  out_hbm.at[pl.ds(r0 + H1, CH - H1)],
                                  sem).wait()

        half = jnp.uint32(0x8000)
        himask = jnp.uint32(0xFFFF0000)

        def compute(inb, outb):
            @pl.loop(0, CH, step=5)
            def _(r0):
                for dr in range(5):
                    r = r0 + dr
                    for g in range(DP // 16):
                        lo = plsc.bitcast(inb[r, pl.ds(16 * g, 16)],
                                          jnp.uint32)
                        hi = plsc.bitcast(inb[r, pl.ds(DP + 16 * g, 16)],
                                          jnp.uint32)
                        outb[r, pl.ds(16 * g, 16)] = (
                            ((lo + half) >> 16) | ((hi + half) & himask))

        fire_in(0, in_a, isem_a)

        @pl.loop(0, NCHK - 1, step=2)
        def _(c):
            fire_in(c + 1, in_b, isem_b)
            drain_in(c, in_a, isem_a)

            @pl.when(c > 0)
            def _():
                drain_out(c - 2, out_a, osem_a)

            compute(in_a, out_a)
            fire_out(c, out_a, osem_a)

            @pl.when(c + 2 < NCHK)
            def _():
                fire_in(c + 2, in_a, isem_a)

            drain_in(c + 1, in_b, isem_b)

            @pl.when(c > 0)
            def _():
                drain_out(c - 1, out_b, osem_b)

            compute(in_b, out_b)
            fire_out(c + 1, out_b, osem_b)

        drain_in(NCHK - 1, in_a, isem_a)
        drain_out(NCHK - 3, out_a, osem_a)
        compute(in_a, out_a)
        fire_out(NCHK - 1, out_a, osem_a)
        drain_out(NCHK - 2, out_b, osem_b)
        drain_out(NCHK - 1, out_a, osem_a)

    return k(item_table)


def kernel(user, item, user_table, item_table, user_bias, item_bias):
    item_flat = item.reshape(B * L).astype(jnp.int32)
    user = user.astype(jnp.int32)
    itab_pk = _pack_sc(item_table)
    return _mf_sc(user, item_flat, user_table, itab_pk, user_bias,
                  item_bias)


# speedup vs baseline: 1.1088x; 1.0171x over previous
import dataclasses
import functools

import jax
import jax.numpy as jnp
from jax import lax
from jax.experimental import pallas as pl
from jax.experimental.pallas import tpu as pltpu
from jax.experimental.pallas import tpu_sc as plsc

B = 4096
L = 200
D = 128
NC = 2
NS = 16
NW = NC * NS
BL = B // NW
S0, S1 = 104, 96
NCHUNK = D // 16


def _take(vec, idx):
    dnums = lax.GatherDimensionNumbers(
        offset_dims=(), collapsed_slice_dims=(0,), start_index_map=(0,))
    return lax.gather(vec, idx[:, None], dnums, slice_sizes=(1,),
                      mode=lax.GatherScatterMode.PROMISE_IN_BOUNDS)


def _lane_sum_bcast(acc):
    iota = lax.iota(jnp.int32, 16)
    for sh in (1, 2, 4, 8):
        acc = acc + _take(acc, lax.bitwise_xor(iota, sh))
    return acc


def _mf_sc(user, item_flat, user_table, item_table, user_bias, item_bias):
    mesh = plsc.VectorSubcoreMesh(core_axis_name="c", subcore_axis_name="s")
    cp = pltpu.CompilerParams()
    if "needs_layout_passes" in pltpu.CompilerParams.__dataclass_fields__:
        cp = dataclasses.replace(cp, needs_layout_passes=False)

    @functools.partial(
        pl.kernel,
        out_type=jax.ShapeDtypeStruct((B, L), jnp.float32),
        mesh=mesh,
        compiler_params=cp,
        scratch_types=[
            pltpu.VMEM((BL,), jnp.int32),
            pltpu.VMEM((BL * L,), jnp.int32),
            pltpu.VMEM((BL, D), jnp.float32),
            pltpu.VMEM((BL,), jnp.float32),
            pltpu.VMEM((L, D), jnp.float32),
            pltpu.VMEM((L, D), jnp.float32),
            pltpu.VMEM((L,), jnp.float32),
            pltpu.VMEM((L,), jnp.float32),
            pltpu.VMEM((BL, L), jnp.float32),
            pltpu.SemaphoreType.DMA,
            pltpu.SemaphoreType.DMA,
            pltpu.SemaphoreType.DMA,
        ],
    )
    def k(user_hbm, item_hbm, utab_hbm, itab_hbm, ubias_hbm, ibias_hbm,
          out_hbm, uidx_v, idx_v, urows_v, ub_v, rows_a, rows_b, ib_a, ib_b,
          out_v, sem_a, sem_b, sem0):
        wid = lax.axis_index("s") * NC + lax.axis_index("c")
        base = wid * BL

        pltpu.sync_copy(user_hbm.at[pl.ds(base, BL)], uidx_v)
        pltpu.sync_copy(item_hbm.at[pl.ds(base * L, BL * L)], idx_v)
        pltpu.async_copy(utab_hbm.at[uidx_v], urows_v, sem0).wait()
        pltpu.async_copy(ubias_hbm.at[uidx_v], ub_v, sem0).wait()

        def idx_views(b):
            o = pl.multiple_of(b * L, 8)
            return (idx_v.at[pl.ds(o, S0)],
                    idx_v.at[pl.ds(pl.multiple_of(b * L + S0, 8), S1)])

        def fire(b, rows, ib, sem):
            i1, i2 = idx_views(b)
            pltpu.async_copy(itab_hbm.at[i1], rows.at[pl.ds(0, S0)], sem)
            pltpu.async_copy(itab_hbm.at[i2], rows.at[pl.ds(S0, S1)], sem)
            pltpu.async_copy(ibias_hbm.at[i1], ib.at[pl.ds(0, S0)], sem)
            pltpu.async_copy(ibias_hbm.at[i2], ib.at[pl.ds(S0, S1)], sem)

        def drain(b, rows, ib, sem):
            i1, i2 = idx_views(b)
            pltpu.make_async_copy(itab_hbm.at[i1], rows.at[pl.ds(0, S0)],
                                  sem).wait()
            pltpu.make_async_copy(itab_hbm.at[i2], rows.at[pl.ds(S0, S1)],
                                  sem).wait()
            pltpu.make_async_copy(ibias_hbm.at[i1], ib.at[pl.ds(0, S0)],
                                  sem).wait()
            pltpu.make_async_copy(ibias_hbm.at[i2], ib.at[pl.ds(S0, S1)],
                                  sem).wait()

        lane_iota = lax.iota(jnp.int32, 16)

        def compute(b, rows, ib):
            u = [urows_v[b, pl.ds(16 * c, 16)] for c in range(NCHUNK)]
            ub_chunk = ub_v[pl.ds((b // 16) * 16, 16)]
            ubs = _take(ub_chunk, jnp.full((16,), lax.rem(b, 16), jnp.int32))

            @pl.loop(0, 13)
            def _(g):
                off = jnp.minimum(16 * g, L - 16)
                out16 = jnp.zeros((16,), jnp.float32)
                for j in range(16):
                    row = off + j
                    acc = rows[row, pl.ds(0, 16)] * u[0]
                    for c in range(1, NCHUNK):
                        acc = acc + rows[row, pl.ds(16 * c, 16)] * u[c]
                    acc = _lane_sum_bcast(acc)
                    out16 = jnp.where(lane_iota == j, acc, out16)
                out16 = out16 + ib[pl.ds(off, 16)] + ubs
                out_v[b, pl.ds(off, 16)] = out16

        fire(0, rows_a, ib_a, sem_a)

        @pl.loop(0, BL, step=2)
        def _(b):
            fire(b + 1, rows_b, ib_b, sem_b)
            drain(b, rows_a, ib_a, sem_a)
            compute(b, rows_a, ib_a)

            @pl.when(b + 2 < BL)
            def _():
                fire(b + 2, rows_a, ib_a, sem_a)

            drain(b + 1, rows_b, ib_b, sem_b)
            compute(b + 1, rows_b, ib_b)

        pltpu.sync_copy(out_v, out_hbm.at[pl.ds(base, BL)])

    return k(user, item_flat, user_table, item_table, user_bias, item_bias)


def kernel(user, item, user_table, item_table, user_bias, item_bias):
    item_flat = item.reshape(B * L).astype(jnp.int32)
    user = user.astype(jnp.int32)
    return _mf_sc(user, item_flat, user_table, item_table, user_bias,
                  item_bias)
